# Initial kernel scaffold; baseline (speedup 1.0000x reference)
import jax
import jax.numpy as jnp
from jax.experimental import pallas as pl


def kernel(x, pos, edge_index, batch, batch_size, params):
    raise NotImplementedError("write your pallas kernel here")


# speedup vs baseline: 1.5072x; 1.5072x over previous
import functools

import numpy as np
import jax
import jax.numpy as jnp
from jax import lax
from jax.experimental import pallas as pl
from jax.experimental.pallas import tpu as pltpu
from jax.experimental.pallas import tpu_sc as plsc

_F32 = jnp.float32
_N = 10000
_E = 320000
_B = 16
_T = 2000
_C = 400
_EPS = 1e-5


def _gn(t, mmat, gamma, beta):
    mean = jnp.dot(t, mmat, preferred_element_type=_F32)
    ex2 = jnp.dot(t * t, mmat, preferred_element_type=_F32)
    var = ex2 - mean * mean
    return (t - mean) * lax.rsqrt(var + _EPS) * gamma + beta


def _enc_body(xin, w1, w2, w3, wa, wb, vecs, m64, ha_ref, hb_ref):
    e = pl.program_id(0)
    x = xin[0]
    b1 = vecs[0, 0]; g1 = vecs[0, 1]; be1 = vecs[0, 2]
    b2 = vecs[0, 3]; g2 = vecs[0, 4]; be2 = vecs[0, 5]
    b3 = vecs[0, 6]
    h = jnp.dot(x, w1[0], preferred_element_type=_F32) + b1
    h = jax.nn.relu(_gn(h, m64[...], g1, be1))
    h = jnp.dot(h, w2[0], preferred_element_type=_F32) + b2
    h = jax.nn.relu(_gn(h, m64[...], g2, be2))
    h = jnp.dot(h, w3[0], preferred_element_type=_F32) + b3
    ca = jnp.dot(h, wa[0], preferred_element_type=_F32)
    cb = jnp.dot(h, wb[0], preferred_element_type=_F32)

    @pl.when(e == 0)
    def _():
        ha_ref[...] = ca
        hb_ref[...] = cb

    @pl.when(e != 0)
    def _():
        ha_ref[...] = ha_ref[...] + ca
        hb_ref[...] = hb_ref[...] + cb


def _encoders(xin, w1, w2, w3, wa, wb, vecs, m64):
    n = xin.shape[1]
    return pl.pallas_call(
        _enc_body,
        grid=(2,),
        in_specs=[
            pl.BlockSpec((1, n, 4), lambda e: (e, 0, 0)),
            pl.BlockSpec((1, 4, 64), lambda e: (e, 0, 0)),
            pl.BlockSpec((1, 64, 64), lambda e: (e, 0, 0)),
            pl.BlockSpec((1, 64, 64), lambda e: (e, 0, 0)),
            pl.BlockSpec((1, 64, 128), lambda e: (e, 0, 0)),
            pl.BlockSpec((1, 64, 128), lambda e: (e, 0, 0)),
            pl.BlockSpec((1, 8, 64), lambda e: (e, 0, 0)),
            pl.BlockSpec((64, 64), lambda e: (0, 0)),
        ],
        out_specs=[
            pl.BlockSpec((n, 128), lambda e: (0, 0)),
            pl.BlockSpec((n, 128), lambda e: (0, 0)),
        ],
        out_shape=[
            jax.ShapeDtypeStruct((n, 128), _F32),
            jax.ShapeDtypeStruct((n, 128), _F32),
        ],
    )(xin, w1, w2, w3, wa, wb, vecs, m64)


def _sc_gather(ha, hb, dst, src):
    info = plsc.get_sparse_core_info()
    nw = info.num_cores * info.num_subcores
    e = dst.shape[0]
    per_w = e // nw
    mesh = plsc.VectorSubcoreMesh(core_axis_name="c", subcore_axis_name="s")

    @functools.partial(
        pl.kernel,
        mesh=mesh,
        out_type=[
            jax.ShapeDtypeStruct((e, 128), _F32),
            jax.ShapeDtypeStruct((e, 128), _F32),
        ],
        scratch_types=[
            pltpu.VMEM((_C,), jnp.int32),
            pltpu.VMEM((_C, 128), _F32),
            pltpu.VMEM((_C,), jnp.int32),
            pltpu.VMEM((_C, 128), _F32),
            pltpu.SemaphoreType.DMA,
            pltpu.SemaphoreType.DMA,
        ],
    )
    def k(ha_hbm, hb_hbm, dst_hbm, src_hbm, ai_hbm, bj_hbm,
          idxa_v, rowsa_v, idxb_v, rowsb_v, sema, semb):
        wid = lax.axis_index("s") * info.num_cores + lax.axis_index("c")
        base = wid * per_w

        def body(ci, _):
            off = base + ci * _C
            pltpu.sync_copy(dst_hbm.at[pl.ds(off, _C)], idxa_v)
            pltpu.sync_copy(src_hbm.at[pl.ds(off, _C)], idxb_v)
            cpa = pltpu.async_copy(ha_hbm.at[idxa_v], rowsa_v, sema)
            cpb = pltpu.async_copy(hb_hbm.at[idxb_v], rowsb_v, semb)
            cpa.wait()
            pltpu.sync_copy(rowsa_v, ai_hbm.at[pl.ds(off, _C)])
            cpb.wait()
            pltpu.sync_copy(rowsb_v, bj_hbm.at[pl.ds(off, _C)])
            return 0

        lax.fori_loop(0, per_w // _C, body, 0)

    return k(ha, hb, dst, src)


def _edge_body(ai, bj, dstr, w2, w3, vecs, m128, agg_ref, acc_ref, ms_ref):
    t = pl.program_id(0)

    @pl.when(t == 0)
    def _():
        acc_ref[...] = jnp.zeros_like(acc_ref)

    g1 = vecs[0]; be1 = vecs[1]; g2 = vecs[2]
    be2 = vecs[3]; g3 = vecs[4]; be3 = vecs[5]
    pre = ai[...] + bj[...]
    m = jax.nn.relu(_gn(pre, m128[...], g1, be1))
    m = jnp.dot(m, w2[...], preferred_element_type=_F32)
    m = jax.nn.relu(_gn(m, m128[...], g2, be2))
    m = jnp.dot(m, w3[...], preferred_element_type=_F32)
    m = jax.nn.relu(_gn(m, m128[...], g3, be3))
    ms_ref[...] = m

    def body(i, _):
        d = dstr[0, 0, i]
        acc_ref[pl.ds(d, 1), :] = jnp.maximum(
            acc_ref[pl.ds(d, 1), :], ms_ref[pl.ds(i, 1), :])
        return 0

    lax.fori_loop(0, _T, body, 0)

    @pl.when(t == pl.num_programs(0) - 1)
    def _():
        agg_ref[...] = acc_ref[...]


def _edge_mlp_aggregate(ai, bj, dstr, w2, w3, vecs, m128):
    return pl.pallas_call(
        _edge_body,
        grid=(_E // _T,),
        in_specs=[
            pl.BlockSpec((_T, 128), lambda t: (t, 0)),
            pl.BlockSpec((_T, 128), lambda t: (t, 0)),
            pl.BlockSpec((1, 1, _T), lambda t: (t, 0, 0),
                         memory_space=pltpu.SMEM),
            pl.BlockSpec((128, 128), lambda t: (0, 0)),
            pl.BlockSpec((128, 128), lambda t: (0, 0)),
            pl.BlockSpec((8, 128), lambda t: (0, 0)),
            pl.BlockSpec((128, 128), lambda t: (0, 0)),
        ],
        out_specs=pl.BlockSpec((_N, 128), lambda t: (0, 0)),
        out_shape=jax.ShapeDtypeStruct((_N, 128), _F32),
        scratch_shapes=[
            pltpu.VMEM((_N, 128), _F32),
            pltpu.VMEM((_T, 128), _F32),
        ],
    )(ai, bj, dstr, w2, w3, vecs, m128)


def _tail_body(agg, bmask, ew1, ew2, evecs, fw1, fw2, fw3, fvecs, m128,
               out_ref):
    h = jnp.dot(agg[...], ew1[...], preferred_element_type=_F32)
    h = jax.nn.relu(_gn(h, m128[...], evecs[0], evecs[1]))
    h = jnp.dot(h, ew2[...], preferred_element_type=_F32)
    h = jax.nn.relu(_gn(h, m128[...], evecs[2], evecs[3]))
    bm = bmask[...]
    rows = []
    for b in range(_B):
        mb = bm[:, b:b + 1]
        rows.append(jnp.max(h * mb, axis=0, keepdims=True))
    gf = jnp.concatenate(rows, axis=0)
    o = jax.nn.relu(
        jnp.dot(gf, fw1[...], preferred_element_type=_F32) + fvecs[0:1])
    o = jax.nn.relu(
        jnp.dot(o, fw2[...], preferred_element_type=_F32) + fvecs[1:2])
    out_ref[...] = (jnp.dot(o, fw3[...], preferred_element_type=_F32)
                    + fvecs[2:3, :6])


def _tail(agg, bmask, ew1, ew2, evecs, fw1, fw2, fw3, fvecs, m128):
    return pl.pallas_call(
        _tail_body,
        out_shape=jax.ShapeDtypeStruct((_B, 6), _F32),
    )(agg, bmask, ew1, ew2, evecs, fw1, fw2, fw3, fvecs, m128)


def kernel(x, pos, edge_index, batch, batch_size, params):
    p = params
    src = edge_index[0]
    dst = edge_index[1]

    m64 = jnp.asarray(np.kron(np.eye(4), np.ones((16, 16)) / 16.0), _F32)
    m128 = jnp.asarray(np.kron(np.eye(8), np.ones((16, 16)) / 16.0), _F32)

    w1 = p['loc_w1']
    wa = w1[:128] - w1[128:]
    wb = w1[128:]

    xin = jnp.stack([x, pos])
    w1s = jnp.stack([p['enc_w1'], p['pos_w1']])
    w2s = jnp.stack([p['enc_w2'], p['pos_w2']])
    w3s = jnp.stack([p['enc_w3'], p['pos_w3']])
    was = jnp.stack([wa[:64], wa[64:]])
    wbs = jnp.stack([wb[:64], wb[64:]])
    zpad = jnp.zeros((64,), _F32)
    evec = lambda pre: jnp.stack([
        p[pre + '_b1'], p[pre + '_g1'], p[pre + '_be1'],
        p[pre + '_b2'], p[pre + '_g2'], p[pre + '_be2'],
        p[pre + '_b3'], zpad])
    vecs64 = jnp.stack([evec('enc'), evec('pos')])

    ha, hb = _encoders(xin, w1s, w2s, w3s, was, wbs, vecs64, m64)
    ai, bj = _sc_gather(ha, hb, dst, src)

    dstr = dst.reshape(_E // _T, 1, _T)
    zpad128 = jnp.zeros((128,), _F32)
    locv = jnp.stack([p['loc_g1'], p['loc_be1'], p['loc_g2'], p['loc_be2'],
                      p['loc_g3'], p['loc_be3'], zpad128, zpad128])
    agg = _edge_mlp_aggregate(ai, bj, dstr, p['loc_w2'], p['loc_w3'],
                              locv, m128)

    bmask = (batch[:, None] == jnp.arange(_B, dtype=jnp.int32)[None, :]
             ).astype(_F32)
    evecs = jnp.stack([p['e2_g1'], p['e2_be1'], p['e2_g2'], p['e2_be2']])
    fb3 = jnp.concatenate([p['fc_b3'], jnp.zeros((122,), _F32)])
    fvecs = jnp.stack([p['fc_b1'], p['fc_b2'], fb3])
    logits = _tail(agg, bmask, p['e2_w1'], p['e2_w2'], evecs,
                   p['fc_w1'], p['fc_w2'], p['fc_w3'], fvecs, m128)
    return logits
